# baseline (speedup 1.0000x reference)
import functools

import jax
import jax.numpy as jnp
from jax import lax
from jax.experimental import pallas as pl
from jax.experimental.pallas import tpu as pltpu
from jax.experimental.pallas import tpu_sc as plsc

_D = 64
_DP = 128
_SCALE = _D ** -0.5
_CHUNK = 64
_NBUF = 8
_LOOK = _NBUF // 2
_TCB = 8192


@functools.lru_cache(maxsize=None)
def _build_prepass(vocab: int):
    def body(tt_ref, out_ref):
        x = tt_ref[...]
        eye = jax.lax.broadcasted_iota(jnp.int32, (_D, _D), 0)
        eye = jnp.where(
            eye == jax.lax.broadcasted_iota(jnp.int32, (_D, _D), 1),
            _SCALE,
            0.0,
        ).astype(jnp.float32)
        y = jax.lax.dot_general(
            x,
            eye,
            (((0,), (0,)), ((), ())),
            precision=jax.lax.Precision.HIGHEST,
        )
        out_ref[:, 0:_D] = y

    return pl.pallas_call(
        body,
        grid=(pl.cdiv(vocab, _TCB),),
        in_specs=[
            pl.BlockSpec((_D, _TCB), lambda i: (0, i)),
        ],
        out_specs=pl.BlockSpec((_TCB, _DP), lambda i: (i, 0)),
        out_shape=jax.ShapeDtypeStruct((vocab, _DP), jnp.float32),
    )


@functools.lru_cache(maxsize=None)
def _build(n_idx: int, vocab: int):
    info = plsc.get_sparse_core_info()
    nw = info.num_cores * info.num_subcores
    per_w = n_idx // nw
    assert n_idx % nw == 0 and per_w % _CHUNK == 0
    n_chunks = per_w // _CHUNK

    mesh = plsc.VectorSubcoreMesh(core_axis_name="c", subcore_axis_name="s")

    scratch = (
        [pltpu.VMEM((per_w,), jnp.int32)]
        + [pltpu.VMEM((_CHUNK, _DP), jnp.float32) for _ in range(_NBUF)]
        + [pltpu.SemaphoreType.DMA for _ in range(2 * _NBUF + 1)]
    )

    @functools.partial(
        pl.kernel,
        out_type=jax.ShapeDtypeStruct((n_idx // _CHUNK, _CHUNK, _DP), jnp.float32),
        mesh=mesh,
        scratch_types=scratch,
        compiler_params=pltpu.CompilerParams(use_tc_tiling_on_sc=True),
    )
    def emb_kernel(table_hbm, x_hbm, out_hbm, *sc):
        idx_v = sc[0]
        gbufs = sc[1 : 1 + _NBUF]
        gsems = sc[1 + _NBUF : 1 + 2 * _NBUF]
        osems = sc[1 + 2 * _NBUF : 1 + 3 * _NBUF]
        isem = sc[1 + 3 * _NBUF]

        wid = lax.axis_index("s") * info.num_cores + lax.axis_index("c")
        base = wid * per_w

        pltpu.async_copy(x_hbm.at[pl.ds(base, per_w)], idx_v, isem).wait()

        def start_gather(c, b):
            pltpu.async_copy(
                table_hbm.at[idx_v.at[pl.ds(c * _CHUNK, _CHUNK)]],
                gbufs[b],
                gsems[b],
            )

        def wait_gather(b):
            pltpu.make_async_copy(
                table_hbm.at[idx_v.at[pl.ds(0, _CHUNK)]], gbufs[b], gsems[b]
            ).wait()

        def start_out(c, b):
            pltpu.async_copy(
                gbufs[b], out_hbm.at[base // _CHUNK + c], osems[b]
            )

        def wait_out(b):
            pltpu.make_async_copy(
                gbufs[b], out_hbm.at[0], osems[b]
            ).wait()

        for b in range(_LOOK):
            start_gather(b, b)

        def round_body(t, carry):
            for b in range(_NBUF):
                c = t * _NBUF + b
                f = (b + _LOOK) % _NBUF
                wait_gather(b)
                start_out(c, b)

                @pl.when(c + _LOOK < n_chunks)
                def _():
                    @pl.when(c >= _LOOK)
                    def _():
                        wait_out(f)

                    start_gather(c + _LOOK, f)
            return carry

        lax.fori_loop(0, n_chunks // _NBUF, round_body, 0)

        for b in range(_NBUF - _LOOK, _NBUF):
            wait_out(b)
        for b in range(_LOOK):
            wait_out(b)

    return emb_kernel


def kernel(x, table):
    n_idx = x.shape[0] * x.shape[1]
    tp = _build_prepass(table.shape[0])(table.T)
    xflat = x.astype(jnp.int32).reshape(n_idx)
    out3 = _build(n_idx, table.shape[0])(tp, xflat)
    out = out3.reshape(n_idx, _DP)[:, :_D]
    return out.reshape(x.shape[0], x.shape[1], _D)


# speedup vs baseline: 1.0009x; 1.0009x over previous
import functools

import jax
import jax.numpy as jnp
from jax import lax
from jax.experimental import pallas as pl
from jax.experimental.pallas import tpu as pltpu
from jax.experimental.pallas import tpu_sc as plsc

_D = 64
_DP = 128
_SCALE = _D ** -0.5
_CHUNK = 80
_NBUF = 8
_LOOK = _NBUF // 2
_TCB = 8192


@functools.lru_cache(maxsize=None)
def _build_prepass(vocab: int):
    def body(tt_ref, out_ref):
        x = tt_ref[...]
        eye = jax.lax.broadcasted_iota(jnp.int32, (_D, _D), 0)
        eye = jnp.where(
            eye == jax.lax.broadcasted_iota(jnp.int32, (_D, _D), 1),
            _SCALE,
            0.0,
        ).astype(jnp.float32)
        y = jax.lax.dot_general(
            x,
            eye,
            (((0,), (0,)), ((), ())),
            precision=jax.lax.Precision.HIGHEST,
        )
        out_ref[:, 0:_D] = y

    return pl.pallas_call(
        body,
        grid=(pl.cdiv(vocab, _TCB),),
        in_specs=[
            pl.BlockSpec((_D, _TCB), lambda i: (0, i)),
        ],
        out_specs=pl.BlockSpec((_TCB, _DP), lambda i: (i, 0)),
        out_shape=jax.ShapeDtypeStruct((vocab, _DP), jnp.float32),
    )


@functools.lru_cache(maxsize=None)
def _build(n_idx: int, vocab: int):
    info = plsc.get_sparse_core_info()
    nw = info.num_cores * info.num_subcores
    per_w = n_idx // nw
    assert n_idx % nw == 0 and per_w % _CHUNK == 0
    n_chunks = per_w // _CHUNK

    mesh = plsc.VectorSubcoreMesh(core_axis_name="c", subcore_axis_name="s")

    scratch = (
        [pltpu.VMEM((per_w,), jnp.int32)]
        + [pltpu.VMEM((_CHUNK, _DP), jnp.float32) for _ in range(_NBUF)]
        + [pltpu.SemaphoreType.DMA for _ in range(2 * _NBUF + 1)]
    )

    @functools.partial(
        pl.kernel,
        out_type=jax.ShapeDtypeStruct((n_idx // _CHUNK, _CHUNK, _DP), jnp.float32),
        mesh=mesh,
        scratch_types=scratch,
        compiler_params=pltpu.CompilerParams(use_tc_tiling_on_sc=True),
    )
    def emb_kernel(table_hbm, x_hbm, out_hbm, *sc):
        idx_v = sc[0]
        gbufs = sc[1 : 1 + _NBUF]
        gsems = sc[1 + _NBUF : 1 + 2 * _NBUF]
        osems = sc[1 + 2 * _NBUF : 1 + 3 * _NBUF]
        isem = sc[1 + 3 * _NBUF]

        wid = lax.axis_index("s") * info.num_cores + lax.axis_index("c")
        base = wid * per_w

        pltpu.async_copy(x_hbm.at[pl.ds(base, per_w)], idx_v, isem).wait()

        def start_gather(c, b):
            pltpu.async_copy(
                table_hbm.at[idx_v.at[pl.ds(c * _CHUNK, _CHUNK)]],
                gbufs[b],
                gsems[b],
            )

        def wait_gather(b):
            pltpu.make_async_copy(
                table_hbm.at[idx_v.at[pl.ds(0, _CHUNK)]], gbufs[b], gsems[b]
            ).wait()

        def start_out(c, b):
            pltpu.async_copy(
                gbufs[b], out_hbm.at[base // _CHUNK + c], osems[b]
            )

        def wait_out(b):
            pltpu.make_async_copy(
                gbufs[b], out_hbm.at[0], osems[b]
            ).wait()

        for b in range(_LOOK):
            start_gather(b, b)

        def round_body(t, carry):
            for b in range(_NBUF):
                c = t * _NBUF + b
                f = (b + _LOOK) % _NBUF
                wait_gather(b)
                start_out(c, b)

                @pl.when(c + _LOOK < n_chunks)
                def _():
                    @pl.when(c >= _LOOK)
                    def _():
                        wait_out(f)

                    start_gather(c + _LOOK, f)
            return carry

        lax.fori_loop(0, n_chunks // _NBUF, round_body, 0)

        for b in range(_NBUF - _LOOK, _NBUF):
            wait_out(b)
        for b in range(_LOOK):
            wait_out(b)

    return emb_kernel


def kernel(x, table):
    n_idx = x.shape[0] * x.shape[1]
    tp = _build_prepass(table.shape[0])(table.T)
    xflat = x.astype(jnp.int32).reshape(n_idx)
    out3 = _build(n_idx, table.shape[0])(tp, xflat)
    out = out3.reshape(n_idx, _DP)[:, :_D]
    return out.reshape(x.shape[0], x.shape[1], _D)
